# baseline (speedup 1.0000x reference)
import functools

import jax
import jax.numpy as jnp
from jax import lax
from jax.experimental import pallas as pl
from jax.experimental.pallas import tpu as pltpu
from jax.experimental.pallas import tpu_sc as plsc

N_NODES = 10000
N_EDGES = 320000
NFEAT = 128
NHID = 128

NC, NS = 2, 16
NW = NC * NS
EPW = N_EDGES // NW
E = 96
NCHUNK = -(-EPW // E)
EPW_PAD = NCHUNK * E
NR = 3
NI = 4
GROUP = 12
N_PAD = 10240
ROWS_PER_TILE = N_PAD // NS
LANES = 16


def _spmm_partials(dst, src, w, x):
    mesh = plsc.VectorSubcoreMesh(
        core_axis_name="c", subcore_axis_name="s", num_cores=NC, num_subcores=NS
    )

    @functools.partial(
        pl.kernel,
        out_type=jax.ShapeDtypeStruct((NC, N_PAD, NFEAT), jnp.float32),
        mesh=mesh,
        scratch_types=[
            [pltpu.VMEM((E, NFEAT), jnp.float32) for _ in range(NR)],
            [pltpu.VMEM((E,), jnp.int32) for _ in range(NI)],
            [pltpu.VMEM((E,), jnp.int32) for _ in range(NI)],
            [pltpu.VMEM((E,), jnp.float32) for _ in range(NI)],
            pltpu.VMEM_SHARED((N_PAD, NFEAT), jnp.float32),
            [pltpu.SemaphoreType.DMA for _ in range(NR)],
            [pltpu.SemaphoreType.DMA for _ in range(NR)],
            [pltpu.SemaphoreType.DMA for _ in range(NI)],
        ],
    )
    def spmm(dst_hbm, src_hbm, w_hbm, x_hbm, out_hbm, rows, si, di, wb, acc,
             gsem, ssem, isem):
        c = lax.axis_index("c")
        s = lax.axis_index("s")
        wid = c * NS + s
        ebase = wid * EPW_PAD

        def idx_copies(i, q):
            off = ebase + i * E
            return (
                pltpu.async_copy(src_hbm.at[pl.ds(off, E)], si[q], isem[q]),
                pltpu.async_copy(dst_hbm.at[pl.ds(off, E)], di[q], isem[q]),
                pltpu.async_copy(w_hbm.at[pl.ds(off, E)], wb[q], isem[q]),
            )

        def wait_idx(q):
            pltpu.make_async_copy(src_hbm.at[pl.ds(0, E)], si[q], isem[q]).wait()
            pltpu.make_async_copy(dst_hbm.at[pl.ds(0, E)], di[q], isem[q]).wait()
            pltpu.make_async_copy(w_hbm.at[pl.ds(0, E)], wb[q], isem[q]).wait()

        def gather(q, r):
            pltpu.async_copy(x_hbm.at[si[q]], rows[r], gsem[r])

        def wait_gather(q, r):
            pltpu.make_async_copy(x_hbm.at[si[q]], rows[r], gsem[r]).wait()

        def scatter(q, r):
            pltpu.async_copy(rows[r], acc.at[di[q]], ssem[r], add=True)

        def wait_scatter(q, r):
            pltpu.make_async_copy(rows[r], acc.at[di[q]], ssem[r]).wait()

        def scale(q, r):
            def grp(g, _):
                wvec16 = wb[q][pl.ds(g * LANES, LANES)]
                for el in range(LANES):
                    wspl = jnp.full((LANES,), wvec16[el], jnp.float32)
                    for j in range(NFEAT // LANES):
                        sl = pl.ds(j * LANES, LANES)
                        e = g * LANES + el
                        rows[r][e, sl] = rows[r][e, sl] * wspl
                return 0

            lax.fori_loop(0, E // LANES, grp, 0)

        zvec = jnp.zeros((LANES,), jnp.float32)

        def zrow(r, _):
            for j in range(NFEAT // LANES):
                rows[0][r, pl.ds(j * LANES, LANES)] = zvec
            return 0

        lax.fori_loop(0, E, zrow, 0)
        for k in range(ROWS_PER_TILE // E):
            pltpu.sync_copy(rows[0], acc.at[pl.ds(s * ROWS_PER_TILE + k * E, E)])
        ZREM = ROWS_PER_TILE - (ROWS_PER_TILE // E) * E
        if ZREM:
            pltpu.sync_copy(
                rows[0].at[pl.ds(0, ZREM)],
                acc.at[pl.ds(s * ROWS_PER_TILE + ROWS_PER_TILE - ZREM, ZREM)],
            )
        plsc.subcore_barrier()

        def step(i, imr, imi, drain=True, fill=True, fire=True):
            nxr = (imr + 1) % NR
            nxi = (imi + 1) % NI
            if drain:
                wait_scatter((imi + NI - (NR - 1)) % NI, nxr)
            if fill:
                idx_copies(i + 2, (imi + 2) % NI)
            wait_gather(imi, imr)
            if fire:
                wait_idx(nxi)
                gather(nxi, nxr)
            scale(imi, imr)
            scatter(imi, imr)

        for d in idx_copies(0, 0):
            d.wait()
        idx_copies(1, 1)
        gather(0, 0)

        HEAD = NR - 1
        for i in range(HEAD):
            step(i, i % NR, i % NI, drain=False)

        ngroups = (NCHUNK - HEAD - 2) // GROUP
        tail0 = HEAD + ngroups * GROUP

        def group(h, _):
            ib = GROUP * h + HEAD
            for k in range(GROUP):
                step(ib + k, (HEAD + k) % NR, (HEAD + k) % NI)
            return 0

        lax.fori_loop(0, ngroups, group, 0)

        for i in range(tail0, NCHUNK):
            step(i, i % NR, i % NI,
                 fill=(i + 2 <= NCHUNK - 1), fire=(i + 1 <= NCHUNK - 1))
        for ch in range(NCHUNK - (NR - 1), NCHUNK):
            wait_scatter(ch % NI, ch % NR)

        plsc.subcore_barrier()
        pltpu.sync_copy(
            acc.at[pl.ds(s * ROWS_PER_TILE, ROWS_PER_TILE)],
            out_hbm.at[c, pl.ds(s * ROWS_PER_TILE, ROWS_PER_TILE)],
        )

    return spmm(dst, src, w, x)


BM = 1000


def _mm_body(p_ref, w_ref, o_ref):
    agg = p_ref[0] + p_ref[1]
    o_ref[...] = jnp.maximum(
        jnp.dot(agg, w_ref[...], preferred_element_type=jnp.float32), 0.0
    )


def _matmul_relu(partials, W):
    return pl.pallas_call(
        _mm_body,
        grid=(N_NODES // BM,),
        in_specs=[
            pl.BlockSpec((NC, BM, NFEAT), lambda i: (0, i, 0)),
            pl.BlockSpec((NFEAT, NHID), lambda i: (0, 0)),
        ],
        out_specs=pl.BlockSpec((BM, NHID), lambda i: (i, 0)),
        out_shape=jax.ShapeDtypeStruct((N_NODES, NHID), jnp.float32),
    )(partials, W)


def _pad_worker(a):
    if EPW_PAD == EPW:
        return a
    return jnp.pad(a.reshape(NW, EPW), ((0, 0), (0, EPW_PAD - EPW))).reshape(-1)


def kernel(edge_index, edge_weight, x, W):
    dst = _pad_worker(edge_index[0])
    src = _pad_worker(edge_index[1])
    w = _pad_worker(edge_weight)
    partials = _spmm_partials(dst, src, w, x)
    return _matmul_relu(partials, W)


# speedup vs baseline: 1.4384x; 1.4384x over previous
import functools

import jax
import jax.numpy as jnp
from jax import lax
from jax.experimental import pallas as pl
from jax.experimental.pallas import tpu as pltpu
from jax.experimental.pallas import tpu_sc as plsc

N_NODES = 10000
N_EDGES = 320000
NFEAT = 128
NHID = 128

NC, NS = 2, 16
NW = NC * NS
EPW = N_EDGES // NW
E = 80
NCHUNK = -(-EPW // E)
EPW_PAD = NCHUNK * E
NR = 3
NI = 4
GROUP = 12
N_PAD = 10240
ROWS_PER_TILE = N_PAD // NS
LANES = 16


def _spmm_partials(dst, src, w, x):
    mesh = plsc.VectorSubcoreMesh(
        core_axis_name="c", subcore_axis_name="s", num_cores=NC, num_subcores=NS
    )

    @functools.partial(
        pl.kernel,
        out_type=jax.ShapeDtypeStruct((NC, N_PAD, NFEAT), jnp.float32),
        mesh=mesh,
        scratch_types=[
            [pltpu.VMEM((E, NFEAT), jnp.float32) for _ in range(NR)],
            [pltpu.VMEM((E,), jnp.int32) for _ in range(NI)],
            [pltpu.VMEM((E,), jnp.int32) for _ in range(NI)],
            [pltpu.VMEM((E,), jnp.float32) for _ in range(NI)],
            pltpu.VMEM_SHARED((N_PAD, NFEAT), jnp.float32),
            [pltpu.SemaphoreType.DMA for _ in range(NR)],
            [pltpu.SemaphoreType.DMA for _ in range(NR)],
            [pltpu.SemaphoreType.DMA for _ in range(NI)],
            pltpu.VMEM((3 * E,), jnp.int32),
        ],
    )
    def spmm(dst_hbm, src_hbm, w_hbm, x_hbm, out_hbm, rows, si, di, wb, acc,
             gsem, ssem, isem, dud):
        c = lax.axis_index("c")
        s = lax.axis_index("s")
        wid = c * NS + s
        ebase = wid * EPW_PAD

        def idx_copies(i, q):
            off = ebase + i * E
            return (
                pltpu.async_copy(src_hbm.at[pl.ds(off, E)], si[q], isem[q]),
                pltpu.async_copy(dst_hbm.at[pl.ds(off, E)], di[q], isem[q]),
                pltpu.async_copy(w_hbm.at[pl.ds(off, E)], wb[q], isem[q]),
            )

        def wait_idx(q):
            pltpu.make_async_copy(
                src_hbm.at[pl.ds(0, 3 * E)], dud, isem[q]
            ).wait()

        def gather(q, r):
            pltpu.async_copy(x_hbm.at[si[q]], rows[r], gsem[r])

        def wait_gather(q, r):
            pltpu.make_async_copy(x_hbm.at[si[q]], rows[r], gsem[r]).wait()

        def scatter(q, r):
            pltpu.async_copy(rows[r], acc.at[di[q]], ssem[r], add=True)

        def wait_scatter(q, r):
            pltpu.make_async_copy(rows[r], acc.at[di[q]], ssem[r]).wait()

        def scale(q, r):
            def grp(g, _):
                wvec16 = wb[q][pl.ds(g * LANES, LANES)]
                for el in range(LANES):
                    wspl = jnp.full((LANES,), wvec16[el], jnp.float32)
                    for j in range(NFEAT // LANES):
                        sl = pl.ds(j * LANES, LANES)
                        e = g * LANES + el
                        rows[r][e, sl] = rows[r][e, sl] * wspl
                return 0

            lax.fori_loop(0, E // LANES, grp, 0)

        zvec = jnp.zeros((LANES,), jnp.float32)

        def zrow(r, _):
            for j in range(NFEAT // LANES):
                rows[0][r, pl.ds(j * LANES, LANES)] = zvec
            return 0

        lax.fori_loop(0, E, zrow, 0)
        for k in range(ROWS_PER_TILE // E):
            pltpu.sync_copy(rows[0], acc.at[pl.ds(s * ROWS_PER_TILE + k * E, E)])
        ZREM = ROWS_PER_TILE - (ROWS_PER_TILE // E) * E
        if ZREM:
            pltpu.sync_copy(
                rows[0].at[pl.ds(0, ZREM)],
                acc.at[pl.ds(s * ROWS_PER_TILE + ROWS_PER_TILE - ZREM, ZREM)],
            )
        plsc.subcore_barrier()

        def step(i, imr, imi, drain=True, fill=True, fire=True):
            nxr = (imr + 1) % NR
            nxi = (imi + 1) % NI
            if drain:
                wait_scatter((imi + NI - (NR - 1)) % NI, nxr)
            if fill:
                idx_copies(i + 2, (imi + 2) % NI)
            wait_gather(imi, imr)
            if fire:
                wait_idx(nxi)
                gather(nxi, nxr)
            scale(imi, imr)
            scatter(imi, imr)

        for d in idx_copies(0, 0):
            d.wait()
        idx_copies(1, 1)
        gather(0, 0)

        HEAD = NR - 1
        for i in range(HEAD):
            step(i, i % NR, i % NI, drain=False)

        ngroups = (NCHUNK - HEAD - 2) // GROUP
        tail0 = HEAD + ngroups * GROUP

        def group(h, _):
            ib = GROUP * h + HEAD
            for k in range(GROUP):
                step(ib + k, (HEAD + k) % NR, (HEAD + k) % NI)
            return 0

        lax.fori_loop(0, ngroups, group, 0)

        for i in range(tail0, NCHUNK):
            step(i, i % NR, i % NI,
                 fill=(i + 2 <= NCHUNK - 1), fire=(i + 1 <= NCHUNK - 1))
        for ch in range(NCHUNK - (NR - 1), NCHUNK):
            wait_scatter(ch % NI, ch % NR)

        plsc.subcore_barrier()
        pltpu.sync_copy(
            acc.at[pl.ds(s * ROWS_PER_TILE, ROWS_PER_TILE)],
            out_hbm.at[c, pl.ds(s * ROWS_PER_TILE, ROWS_PER_TILE)],
        )

    return spmm(dst, src, w, x)


BM = 1000


def _mm_body(p_ref, w_ref, o_ref):
    agg = p_ref[0] + p_ref[1]
    o_ref[...] = jnp.maximum(
        jnp.dot(agg, w_ref[...], preferred_element_type=jnp.float32), 0.0
    )


def _matmul_relu(partials, W):
    return pl.pallas_call(
        _mm_body,
        grid=(N_NODES // BM,),
        in_specs=[
            pl.BlockSpec((NC, BM, NFEAT), lambda i: (0, i, 0)),
            pl.BlockSpec((NFEAT, NHID), lambda i: (0, 0)),
        ],
        out_specs=pl.BlockSpec((BM, NHID), lambda i: (i, 0)),
        out_shape=jax.ShapeDtypeStruct((N_NODES, NHID), jnp.float32),
    )(partials, W)


def _pad_worker(a):
    if EPW_PAD == EPW:
        return a
    return jnp.pad(a.reshape(NW, EPW), ((0, 0), (0, EPW_PAD - EPW))).reshape(-1)


def kernel(edge_index, edge_weight, x, W):
    dst = _pad_worker(edge_index[0])
    src = _pad_worker(edge_index[1])
    w = _pad_worker(edge_weight)
    partials = _spmm_partials(dst, src, w, x)
    return _matmul_relu(partials, W)
